# baseline (speedup 1.0000x reference)
import jax
import jax.numpy as jnp
from jax import lax
from jax.experimental import pallas as pl
from jax.experimental.pallas import tpu as pltpu, tpu_sc as plsc
import functools

NC = 2
NS = 16
NW = NC * NS
B_TOTAL = 16384
DIM = 64
NUM_ENTS_C = 1000000
NUM_RELS_C = 1000
BPW = B_TOTAL // NW
CHUNK = 128
NCHUNK = BPW // CHUNK
GROUPS = BPW // 16
BLK = 32768

E_SC = 196608
SPW = E_SC // NW
CENT = 512
NCH = SPW // CENT
E_SC_BLKS = E_SC // BLK


def _proj_body(wt_ref, tab_ref, rel_ref, b_ref, o0_ref, o1_ref,
               r0_ref, r1_ref):
    res = jnp.dot(wt_ref[...], tab_ref[...],
                  preferred_element_type=jnp.float32)
    o0_ref[...] = res[0]
    o1_ref[...] = res[1]

    @pl.when(pl.program_id(0) == 0)
    def _():
        rp = jnp.dot(wt_ref[...], rel_ref[...],
                     preferred_element_type=jnp.float32) + b_ref[...]
        r0_ref[...] = rp[0]
        r1_ref[...] = rp[1]


def _sc_proj_body(tab_h, ws_h, p0_h, p1_h,
                  ws_v, tab_a, tab_b, ov0, ov1, sem):
    c = lax.axis_index("c")
    s = lax.axis_index("s")
    wid = s * NC + c
    base = wid * SPW

    pltpu.sync_copy(ws_h, ws_v)

    bufs = (tab_a, tab_b)
    cps = [None] * NCH
    cps[0] = pltpu.async_copy(
        tab_h.at[:, pl.ds(base, CENT)], bufs[0], sem)
    for ci in range(NCH):
        if ci + 1 < NCH:
            cps[ci + 1] = pltpu.async_copy(
                tab_h.at[:, pl.ds(base + (ci + 1) * CENT, CENT)],
                bufs[(ci + 1) % 2], sem)
        cps[ci].wait()
        tab = bufs[ci % 2]
        for gb in range(CENT // 128):
            def d_step(d, accs):
                w0 = ws_v[pl.ds(d * 16, 16)]
                w1 = ws_v[pl.ds(DIM * 16 + d * 16, 16)]
                out = []
                for g8 in range(8):
                    v = tab[d, pl.ds(gb * 128 + g8 * 16, 16)]
                    out.append(accs[g8] + v * w0)
                for g8 in range(8):
                    v = tab[d, pl.ds(gb * 128 + g8 * 16, 16)]
                    out.append(accs[8 + g8] + v * w1)
                return tuple(out)

            zero = jnp.zeros((16,), jnp.float32)
            accs = lax.fori_loop(0, DIM, d_step, (zero,) * 16)
            for g8 in range(8):
                sl = pl.ds(ci * CENT + gb * 128 + g8 * 16, 16)
                ov0[sl] = accs[g8]
                ov1[sl] = accs[8 + g8]

    pltpu.sync_copy(ov0, p0_h.at[pl.ds(base, SPW)])
    pltpu.sync_copy(ov1, p1_h.at[pl.ds(base, SPW)])


def _sc_gather_body(e0_h, e1_h, p0_h, p1_h, r0_h, r1_h,
                    head_h, rel_h, tail_h, o0_h, o1_h,
                    idx_h, idx_r, idx_t, idx_htc, idx_hsc, idx_ttc, idx_tsc,
                    hp0t, hp1t, hp0s, hp1s, tp0t, tp1t, tp0s, tp1s, rp0, rp1,
                    ov0, ov1, sem):
    c = lax.axis_index("c")
    s = lax.axis_index("s")
    wid = s * NC + c
    base = wid * BPW

    pltpu.sync_copy(head_h.at[wid], idx_h)
    pltpu.sync_copy(rel_h.at[wid], idx_r)
    pltpu.sync_copy(tail_h.at[wid], idx_t)

    for k in range(NCHUNK):
        for i in range(CHUNK // 16):
            sl = pl.ds(i * 16, 16)
            vh = idx_h[k, sl]
            idx_htc[k, sl] = jnp.maximum(vh, E_SC)
            idx_hsc[k, sl] = jnp.minimum(vh, E_SC - 1)
            vt = idx_t[k, sl]
            idx_ttc[k, sl] = jnp.maximum(vt, E_SC)
            idx_tsc[k, sl] = jnp.minimum(vt, E_SC - 1)

    cps = []
    for k in range(NCHUNK):
        sl = pl.ds(k * CHUNK, CHUNK)
        cps.append(pltpu.async_copy(e0_h.at[idx_htc.at[k]], hp0t.at[sl], sem))
        cps.append(pltpu.async_copy(e1_h.at[idx_htc.at[k]], hp1t.at[sl], sem))
        cps.append(pltpu.async_copy(p0_h.at[idx_hsc.at[k]], hp0s.at[sl], sem))
        cps.append(pltpu.async_copy(p1_h.at[idx_hsc.at[k]], hp1s.at[sl], sem))
        cps.append(pltpu.async_copy(e0_h.at[idx_ttc.at[k]], tp0t.at[sl], sem))
        cps.append(pltpu.async_copy(e1_h.at[idx_ttc.at[k]], tp1t.at[sl], sem))
        cps.append(pltpu.async_copy(p0_h.at[idx_tsc.at[k]], tp0s.at[sl], sem))
        cps.append(pltpu.async_copy(p1_h.at[idx_tsc.at[k]], tp1s.at[sl], sem))
        cps.append(pltpu.async_copy(r0_h.at[idx_r.at[k]], rp0.at[sl], sem))
        cps.append(pltpu.async_copy(r1_h.at[idx_r.at[k]], rp1.at[sl], sem))
    for cp in cps:
        cp.wait()

    for g in range(GROUPS):
        k, off = g // 8, (g % 8) * 16
        isl = pl.ds(off, 16)
        sl = pl.ds(g * 16, 16)
        hm = idx_h[k, isl] >= E_SC
        tm = idx_t[k, isl] >= E_SC
        h0 = jnp.where(hm, hp0t[sl], hp0s[sl])
        h1 = jnp.where(hm, hp1t[sl], hp1s[sl])
        t0 = jnp.where(tm, tp0t[sl], tp0s[sl])
        t1 = jnp.where(tm, tp1t[sl], tp1s[sl])
        a0 = h0 + rp0[sl] - t0
        a1 = h1 + rp1[sl] - t1
        ov0[sl] = 1.0 / (1.0 + jnp.exp(-a0))
        ov1[sl] = 1.0 / (1.0 + jnp.exp(-a1))

    pltpu.sync_copy(ov0, o0_h.at[pl.ds(base, BPW)])
    pltpu.sync_copy(ov1, o1_h.at[pl.ds(base, BPW)])


@jax.jit
def _run(head3, rel3, tail3, ent_emb, rel_emb, lin_W, lin_b):
    wt = lin_W.T
    ent_t = ent_emb.T
    rel_t = rel_emb.T
    b2 = lin_b.reshape(2, 1)
    wspl = jnp.repeat(wt.reshape(-1), 16)

    grid = (NUM_ENTS_C - E_SC + BLK - 1) // BLK
    e0, e1, r0, r1 = pl.pallas_call(
        _proj_body,
        grid=(grid,),
        in_specs=[
            pl.BlockSpec((2, DIM), lambda i: (0, 0)),
            pl.BlockSpec((DIM, BLK), lambda i: (0, i + E_SC_BLKS)),
            pl.BlockSpec((DIM, NUM_RELS_C), lambda i: (0, 0)),
            pl.BlockSpec((2, 1), lambda i: (0, 0)),
        ],
        out_specs=[
            pl.BlockSpec((BLK,), lambda i: (i + E_SC_BLKS,)),
            pl.BlockSpec((BLK,), lambda i: (i + E_SC_BLKS,)),
            pl.BlockSpec((NUM_RELS_C,), lambda i: (0,)),
            pl.BlockSpec((NUM_RELS_C,), lambda i: (0,)),
        ],
        out_shape=[
            jax.ShapeDtypeStruct((NUM_ENTS_C,), jnp.float32),
            jax.ShapeDtypeStruct((NUM_ENTS_C,), jnp.float32),
            jax.ShapeDtypeStruct((NUM_RELS_C,), jnp.float32),
            jax.ShapeDtypeStruct((NUM_RELS_C,), jnp.float32),
        ],
        compiler_params=pltpu.CompilerParams(vmem_limit_bytes=60000000),
    )(wt, ent_t, rel_t, b2)

    mesh = plsc.VectorSubcoreMesh(core_axis_name="c", subcore_axis_name="s")

    proj_sc = pl.kernel(
        _sc_proj_body,
        out_type=(
            jax.ShapeDtypeStruct((E_SC,), jnp.float32),
            jax.ShapeDtypeStruct((E_SC,), jnp.float32),
        ),
        mesh=mesh,
        compiler_params=pltpu.CompilerParams(needs_layout_passes=False),
        scratch_types=[
            pltpu.VMEM((2 * DIM * 16,), jnp.float32),
            pltpu.VMEM((DIM, CENT), jnp.float32),
            pltpu.VMEM((DIM, CENT), jnp.float32),
            pltpu.VMEM((SPW,), jnp.float32),
            pltpu.VMEM((SPW,), jnp.float32),
            pltpu.SemaphoreType.DMA,
        ],
        name="transe_sc_proj",
    )
    p0, p1 = proj_sc(ent_t, wspl)

    gather_sc = pl.kernel(
        _sc_gather_body,
        out_type=(
            jax.ShapeDtypeStruct((B_TOTAL,), jnp.float32),
            jax.ShapeDtypeStruct((B_TOTAL,), jnp.float32),
        ),
        mesh=mesh,
        compiler_params=pltpu.CompilerParams(needs_layout_passes=False),
        scratch_types=(
            [pltpu.VMEM((NCHUNK, CHUNK), jnp.int32)] * 7
            + [pltpu.VMEM((BPW,), jnp.float32)] * 12
            + [pltpu.SemaphoreType.DMA]
        ),
        name="transe_sc_gather",
    )
    o0, o1 = gather_sc(e0, e1, p0, p1, r0, r1, head3, rel3, tail3)
    return jnp.stack([o0, o1], axis=1)


def kernel(head, rel, tail, ent_emb, rel_emb, lin_W, lin_b):
    head3 = head.astype(jnp.int32).reshape(NW, NCHUNK, CHUNK)
    rel3 = rel.astype(jnp.int32).reshape(NW, NCHUNK, CHUNK)
    tail3 = tail.astype(jnp.int32).reshape(NW, NCHUNK, CHUNK)
    return _run(head3, rel3, tail3, ent_emb, rel_emb, lin_W, lin_b)


# speedup vs baseline: 1.8889x; 1.8889x over previous
import jax
import jax.numpy as jnp
from jax import lax
from jax.experimental import pallas as pl
from jax.experimental.pallas import tpu as pltpu, tpu_sc as plsc
import functools

NC = 2
NS = 16
NW = NC * NS
B_TOTAL = 16384
DIM = 64
NUM_ENTS_C = 1000000
NUM_RELS_C = 1000
BPW = B_TOTAL // NW
CHUNK = 128
NCHUNK = BPW // CHUNK
GROUPS = BPW // 16
BLK = 32768

E_SC = 196608
SPW = E_SC // NW
CENT = 512
NCH = SPW // CENT
E_SC_BLKS = E_SC // BLK


def _proj_body(wt_ref, tab_ref, rel_ref, b_ref, o0_ref, o1_ref,
               r0_ref, r1_ref):
    res = jnp.dot(wt_ref[...], tab_ref[...],
                  preferred_element_type=jnp.float32)
    o0_ref[...] = res[0]
    o1_ref[...] = res[1]

    @pl.when(pl.program_id(0) == 0)
    def _():
        rp = jnp.dot(wt_ref[...], rel_ref[...],
                     preferred_element_type=jnp.float32) + b_ref[...]
        r0_ref[...] = rp[0]
        r1_ref[...] = rp[1]


def _sc_proj_body(tab_h, ws_h, p0_h, p1_h,
                  ws_v, tab_a, tab_b, ov0, ov1, sem):
    c = lax.axis_index("c")
    s = lax.axis_index("s")
    wid = s * NC + c
    base = wid * SPW

    pltpu.sync_copy(ws_h, ws_v)

    bufs = (tab_a, tab_b)
    cps = [None] * NCH
    cps[0] = pltpu.async_copy(
        tab_h.at[:, pl.ds(base, CENT)], bufs[0], sem)
    for ci in range(NCH):
        if ci + 1 < NCH:
            cps[ci + 1] = pltpu.async_copy(
                tab_h.at[:, pl.ds(base + (ci + 1) * CENT, CENT)],
                bufs[(ci + 1) % 2], sem)
        cps[ci].wait()
        tab = bufs[ci % 2]
        for gb in range(CENT // 128):
            def d_step(d, accs):
                w0 = ws_v[pl.ds(d * 16, 16)]
                w1 = ws_v[pl.ds(DIM * 16 + d * 16, 16)]
                out = []
                for g8 in range(8):
                    v = tab[d, pl.ds(gb * 128 + g8 * 16, 16)]
                    out.append(accs[g8] + v * w0)
                for g8 in range(8):
                    v = tab[d, pl.ds(gb * 128 + g8 * 16, 16)]
                    out.append(accs[8 + g8] + v * w1)
                return tuple(out)

            zero = jnp.zeros((16,), jnp.float32)
            accs = lax.fori_loop(0, DIM, d_step, (zero,) * 16)
            for g8 in range(8):
                sl = pl.ds(ci * CENT + gb * 128 + g8 * 16, 16)
                ov0[sl] = accs[g8]
                ov1[sl] = accs[8 + g8]

    pltpu.sync_copy(ov0, p0_h.at[pl.ds(base, SPW)])
    pltpu.sync_copy(ov1, p1_h.at[pl.ds(base, SPW)])


def _sc_gather_body(e0_h, e1_h, r0_h, r1_h, head_h, rel_h, tail_h, o0_h, o1_h,
                    idx_h, idx_r, idx_t, hp0, hp1, tp0, tp1, rp0, rp1,
                    ov0, ov1, sem):
    c = lax.axis_index("c")
    s = lax.axis_index("s")
    wid = s * NC + c
    base = wid * BPW

    pltpu.sync_copy(head_h.at[wid], idx_h)
    pltpu.sync_copy(rel_h.at[wid], idx_r)
    pltpu.sync_copy(tail_h.at[wid], idx_t)

    cps = []
    for k in range(NCHUNK):
        sl = pl.ds(k * CHUNK, CHUNK)
        cps.append(pltpu.async_copy(e0_h.at[idx_h.at[k]], hp0.at[sl], sem))
        cps.append(pltpu.async_copy(e1_h.at[idx_h.at[k]], hp1.at[sl], sem))
        cps.append(pltpu.async_copy(e0_h.at[idx_t.at[k]], tp0.at[sl], sem))
        cps.append(pltpu.async_copy(e1_h.at[idx_t.at[k]], tp1.at[sl], sem))
        cps.append(pltpu.async_copy(r0_h.at[idx_r.at[k]], rp0.at[sl], sem))
        cps.append(pltpu.async_copy(r1_h.at[idx_r.at[k]], rp1.at[sl], sem))
    for cp in cps:
        cp.wait()

    def group_step(g, carry):
        sl = pl.ds(pl.multiple_of(g * 16, 16), 16)
        a0 = hp0[sl] + rp0[sl] - tp0[sl]
        a1 = hp1[sl] + rp1[sl] - tp1[sl]
        ov0[sl] = 1.0 / (1.0 + jnp.exp(-a0))
        ov1[sl] = 1.0 / (1.0 + jnp.exp(-a1))
        return carry

    lax.fori_loop(0, GROUPS, group_step, 0)
    pltpu.sync_copy(ov0, o0_h.at[pl.ds(base, BPW)])
    pltpu.sync_copy(ov1, o1_h.at[pl.ds(base, BPW)])


@jax.jit
def _run(head3, rel3, tail3, ent_emb, rel_emb, lin_W, lin_b):
    wt = lin_W.T
    ent_t = ent_emb.T
    rel_t = rel_emb.T
    b2 = lin_b.reshape(2, 1)
    wspl = jnp.repeat(wt.reshape(-1), 16)

    grid = (NUM_ENTS_C - E_SC + BLK - 1) // BLK
    e0, e1, r0, r1 = pl.pallas_call(
        _proj_body,
        grid=(grid,),
        in_specs=[
            pl.BlockSpec((2, DIM), lambda i: (0, 0)),
            pl.BlockSpec((DIM, BLK), lambda i: (0, i + E_SC_BLKS)),
            pl.BlockSpec((DIM, NUM_RELS_C), lambda i: (0, 0)),
            pl.BlockSpec((2, 1), lambda i: (0, 0)),
        ],
        out_specs=[
            pl.BlockSpec((BLK,), lambda i: (i,)),
            pl.BlockSpec((BLK,), lambda i: (i,)),
            pl.BlockSpec((NUM_RELS_C,), lambda i: (0,)),
            pl.BlockSpec((NUM_RELS_C,), lambda i: (0,)),
        ],
        out_shape=[
            jax.ShapeDtypeStruct((NUM_ENTS_C - E_SC,), jnp.float32),
            jax.ShapeDtypeStruct((NUM_ENTS_C - E_SC,), jnp.float32),
            jax.ShapeDtypeStruct((NUM_RELS_C,), jnp.float32),
            jax.ShapeDtypeStruct((NUM_RELS_C,), jnp.float32),
        ],
        compiler_params=pltpu.CompilerParams(vmem_limit_bytes=60000000),
    )(wt, ent_t, rel_t, b2)

    mesh = plsc.VectorSubcoreMesh(core_axis_name="c", subcore_axis_name="s")

    proj_sc = pl.kernel(
        _sc_proj_body,
        out_type=(
            jax.ShapeDtypeStruct((E_SC,), jnp.float32),
            jax.ShapeDtypeStruct((E_SC,), jnp.float32),
        ),
        mesh=mesh,
        compiler_params=pltpu.CompilerParams(needs_layout_passes=False),
        scratch_types=[
            pltpu.VMEM((2 * DIM * 16,), jnp.float32),
            pltpu.VMEM((DIM, CENT), jnp.float32),
            pltpu.VMEM((DIM, CENT), jnp.float32),
            pltpu.VMEM((SPW,), jnp.float32),
            pltpu.VMEM((SPW,), jnp.float32),
            pltpu.SemaphoreType.DMA,
        ],
        name="transe_sc_proj",
    )
    p0, p1 = proj_sc(ent_t, wspl)

    pe0 = jnp.concatenate([p0, e0])
    pe1 = jnp.concatenate([p1, e1])

    gather_sc = pl.kernel(
        _sc_gather_body,
        out_type=(
            jax.ShapeDtypeStruct((B_TOTAL,), jnp.float32),
            jax.ShapeDtypeStruct((B_TOTAL,), jnp.float32),
        ),
        mesh=mesh,
        compiler_params=pltpu.CompilerParams(needs_layout_passes=False),
        scratch_types=(
            [pltpu.VMEM((NCHUNK, CHUNK), jnp.int32)] * 3
            + [pltpu.VMEM((BPW,), jnp.float32)] * 8
            + [pltpu.SemaphoreType.DMA]
        ),
        name="transe_sc_gather",
    )
    o0, o1 = gather_sc(pe0, pe1, r0, r1, head3, rel3, tail3)
    return jnp.stack([o0, o1], axis=1)


def kernel(head, rel, tail, ent_emb, rel_emb, lin_W, lin_b):
    head3 = head.astype(jnp.int32).reshape(NW, NCHUNK, CHUNK)
    rel3 = rel.astype(jnp.int32).reshape(NW, NCHUNK, CHUNK)
    tail3 = tail.astype(jnp.int32).reshape(NW, NCHUNK, CHUNK)
    return _run(head3, rel3, tail3, ent_emb, rel_emb, lin_W, lin_b)


# speedup vs baseline: 1.9022x; 1.0070x over previous
import jax
import jax.numpy as jnp
from jax import lax
from jax.experimental import pallas as pl
from jax.experimental.pallas import tpu as pltpu, tpu_sc as plsc
import functools

NC = 2
NS = 16
NW = NC * NS
B_TOTAL = 16384
DIM = 64
NUM_ENTS_C = 1000000
NUM_RELS_C = 1000
BPW = B_TOTAL // NW
CHUNK = 128
NCHUNK = BPW // CHUNK
GROUPS = BPW // 16
BLK = 32768


def _proj_body(wt_ref, tab_ref, rel_ref, b_ref, o0_ref, o1_ref,
               r0_ref, r1_ref):
    res = jnp.dot(wt_ref[...], tab_ref[...],
                  preferred_element_type=jnp.float32)
    o0_ref[...] = res[0]
    o1_ref[...] = res[1]

    @pl.when(pl.program_id(0) == 0)
    def _():
        rp = jnp.dot(wt_ref[...], rel_ref[...],
                     preferred_element_type=jnp.float32) + b_ref[...]
        r0_ref[...] = rp[0]
        r1_ref[...] = rp[1]


def _sc_body(e0_h, e1_h, r0_h, r1_h, head_h, rel_h, tail_h, o0_h, o1_h,
             idx_h, idx_r, idx_t, hp0, hp1, tp0, tp1, rp0, rp1,
             ov0, ov1, sem):
    c = lax.axis_index("c")
    s = lax.axis_index("s")
    wid = s * NC + c
    base = wid * BPW

    pltpu.sync_copy(head_h.at[wid], idx_h)
    pltpu.sync_copy(rel_h.at[wid], idx_r)
    pltpu.sync_copy(tail_h.at[wid], idx_t)

    cps = [
        pltpu.async_copy(e0_h.at[idx_h], hp0, sem),
        pltpu.async_copy(e1_h.at[idx_h], hp1, sem),
        pltpu.async_copy(e0_h.at[idx_t], tp0, sem),
        pltpu.async_copy(e1_h.at[idx_t], tp1, sem),
        pltpu.async_copy(r0_h.at[idx_r], rp0, sem),
        pltpu.async_copy(r1_h.at[idx_r], rp1, sem),
    ]
    for cp in cps:
        cp.wait()

    def group_step(g, carry):
        sl = pl.ds(pl.multiple_of(g * 16, 16), 16)
        a0 = hp0[sl] + rp0[sl] - tp0[sl]
        a1 = hp1[sl] + rp1[sl] - tp1[sl]
        ov0[sl] = 1.0 / (1.0 + jnp.exp(-a0))
        ov1[sl] = 1.0 / (1.0 + jnp.exp(-a1))
        return carry

    lax.fori_loop(0, GROUPS, group_step, 0)
    pltpu.sync_copy(ov0, o0_h.at[pl.ds(base, BPW)])
    pltpu.sync_copy(ov1, o1_h.at[pl.ds(base, BPW)])


@jax.jit
def _run(head3, rel3, tail3, ent_emb, rel_emb, lin_W, lin_b):
    wt = lin_W.T
    ent_t = ent_emb.T
    rel_t = rel_emb.T
    b2 = lin_b.reshape(2, 1)

    grid = (NUM_ENTS_C + BLK - 1) // BLK
    e0, e1, r0, r1 = pl.pallas_call(
        _proj_body,
        grid=(grid,),
        in_specs=[
            pl.BlockSpec((2, DIM), lambda i: (0, 0)),
            pl.BlockSpec((DIM, BLK), lambda i: (0, i)),
            pl.BlockSpec((DIM, NUM_RELS_C), lambda i: (0, 0)),
            pl.BlockSpec((2, 1), lambda i: (0, 0)),
        ],
        out_specs=[
            pl.BlockSpec((BLK,), lambda i: (i,)),
            pl.BlockSpec((BLK,), lambda i: (i,)),
            pl.BlockSpec((NUM_RELS_C,), lambda i: (0,)),
            pl.BlockSpec((NUM_RELS_C,), lambda i: (0,)),
        ],
        out_shape=[
            jax.ShapeDtypeStruct((NUM_ENTS_C,), jnp.float32),
            jax.ShapeDtypeStruct((NUM_ENTS_C,), jnp.float32),
            jax.ShapeDtypeStruct((NUM_RELS_C,), jnp.float32),
            jax.ShapeDtypeStruct((NUM_RELS_C,), jnp.float32),
        ],
        compiler_params=pltpu.CompilerParams(vmem_limit_bytes=60000000),
    )(wt, ent_t, rel_t, b2)

    mesh = plsc.VectorSubcoreMesh(core_axis_name="c", subcore_axis_name="s")
    f = pl.kernel(
        _sc_body,
        out_type=(
            jax.ShapeDtypeStruct((B_TOTAL,), jnp.float32),
            jax.ShapeDtypeStruct((B_TOTAL,), jnp.float32),
        ),
        mesh=mesh,
        compiler_params=pltpu.CompilerParams(needs_layout_passes=False),
        scratch_types=[
            pltpu.VMEM((BPW,), jnp.int32),
            pltpu.VMEM((BPW,), jnp.int32),
            pltpu.VMEM((BPW,), jnp.int32),
            pltpu.VMEM((BPW,), jnp.float32),
            pltpu.VMEM((BPW,), jnp.float32),
            pltpu.VMEM((BPW,), jnp.float32),
            pltpu.VMEM((BPW,), jnp.float32),
            pltpu.VMEM((BPW,), jnp.float32),
            pltpu.VMEM((BPW,), jnp.float32),
            pltpu.VMEM((BPW,), jnp.float32),
            pltpu.VMEM((BPW,), jnp.float32),
            pltpu.SemaphoreType.DMA,
        ],
        name="transe_sc",
    )
    o0, o1 = f(e0, e1, r0, r1, head3, rel3, tail3)
    return jnp.stack([o0, o1], axis=1)


def kernel(head, rel, tail, ent_emb, rel_emb, lin_W, lin_b):
    head3 = head.astype(jnp.int32).reshape(NW, BPW)
    rel3 = rel.astype(jnp.int32).reshape(NW, BPW)
    tail3 = tail.astype(jnp.int32).reshape(NW, BPW)
    return _run(head3, rel3, tail3, ent_emb, rel_emb, lin_W, lin_b)


# speedup vs baseline: 2.0108x; 1.0571x over previous
import jax
import jax.numpy as jnp
from jax import lax
from jax.experimental import pallas as pl
from jax.experimental.pallas import tpu as pltpu, tpu_sc as plsc
import functools

NC = 2
NS = 16
NW = NC * NS
B_TOTAL = 16384
DIM = 64
NUM_ENTS_C = 1000000
NUM_RELS_C = 1000
BPW = B_TOTAL // NW
CHUNK = 128
NCHUNK = BPW // CHUNK
GROUPS = BPW // 16
BLK = 32768


def _proj_body(wt_ref, tab_ref, rel_ref, b_ref, o0_ref, o1_ref,
               r0_ref, r1_ref):
    res = jnp.dot(wt_ref[...], tab_ref[...],
                  preferred_element_type=jnp.float32)
    o0_ref[...] = res[0]
    o1_ref[...] = res[1]

    @pl.when(pl.program_id(0) == 0)
    def _():
        rp = jnp.dot(wt_ref[...], rel_ref[...],
                     preferred_element_type=jnp.float32) + b_ref[...]
        r0_ref[...] = rp[0]
        r1_ref[...] = rp[1]


def _sc_body(e0_h, e1_h, r0_h, r1_h, head_h, rel_h, tail_h, o0_h, o1_h,
             idx_h, idx_r, idx_t, hp0, hp1, tp0, tp1, rp0, rp1,
             ov0, ov1, sem):
    c = lax.axis_index("c")
    s = lax.axis_index("s")
    wid = s * NC + c
    base = wid * BPW

    pltpu.sync_copy(head_h.at[wid], idx_h)
    pltpu.sync_copy(rel_h.at[wid], idx_r)
    pltpu.sync_copy(tail_h.at[wid], idx_t)

    cps = []
    for k in range(NCHUNK):
        sl = pl.ds(k * CHUNK, CHUNK)
        cps.append(pltpu.async_copy(e0_h.at[idx_h.at[k]], hp0.at[sl], sem))
        cps.append(pltpu.async_copy(e1_h.at[idx_h.at[k]], hp1.at[sl], sem))
        cps.append(pltpu.async_copy(e0_h.at[idx_t.at[k]], tp0.at[sl], sem))
        cps.append(pltpu.async_copy(e1_h.at[idx_t.at[k]], tp1.at[sl], sem))
        cps.append(pltpu.async_copy(r0_h.at[idx_r.at[k]], rp0.at[sl], sem))
        cps.append(pltpu.async_copy(r1_h.at[idx_r.at[k]], rp1.at[sl], sem))
    for cp in cps:
        cp.wait()

    def group_step(g, carry):
        sl = pl.ds(pl.multiple_of(g * 16, 16), 16)
        a0 = hp0[sl] + rp0[sl] - tp0[sl]
        a1 = hp1[sl] + rp1[sl] - tp1[sl]
        ov0[sl] = 1.0 / (1.0 + jnp.exp(-a0))
        ov1[sl] = 1.0 / (1.0 + jnp.exp(-a1))
        return carry

    lax.fori_loop(0, GROUPS, group_step, 0)
    pltpu.sync_copy(ov0, o0_h.at[pl.ds(base, BPW)])
    pltpu.sync_copy(ov1, o1_h.at[pl.ds(base, BPW)])


@jax.jit
def _run(head3, rel3, tail3, ent_emb, rel_emb, lin_W, lin_b):
    wt = lin_W.T
    ent_t = ent_emb.T
    rel_t = rel_emb.T
    b2 = lin_b.reshape(2, 1)

    grid = (NUM_ENTS_C + BLK - 1) // BLK
    e0, e1, r0, r1 = pl.pallas_call(
        _proj_body,
        grid=(grid,),
        in_specs=[
            pl.BlockSpec((2, DIM), lambda i: (0, 0)),
            pl.BlockSpec((DIM, BLK), lambda i: (0, i)),
            pl.BlockSpec((DIM, NUM_RELS_C), lambda i: (0, 0)),
            pl.BlockSpec((2, 1), lambda i: (0, 0)),
        ],
        out_specs=[
            pl.BlockSpec((BLK,), lambda i: (i,)),
            pl.BlockSpec((BLK,), lambda i: (i,)),
            pl.BlockSpec((NUM_RELS_C,), lambda i: (0,)),
            pl.BlockSpec((NUM_RELS_C,), lambda i: (0,)),
        ],
        out_shape=[
            jax.ShapeDtypeStruct((NUM_ENTS_C,), jnp.float32),
            jax.ShapeDtypeStruct((NUM_ENTS_C,), jnp.float32),
            jax.ShapeDtypeStruct((NUM_RELS_C,), jnp.float32),
            jax.ShapeDtypeStruct((NUM_RELS_C,), jnp.float32),
        ],
        compiler_params=pltpu.CompilerParams(vmem_limit_bytes=60000000),
    )(wt, ent_t, rel_t, b2)

    mesh = plsc.VectorSubcoreMesh(core_axis_name="c", subcore_axis_name="s")
    f = pl.kernel(
        _sc_body,
        out_type=(
            jax.ShapeDtypeStruct((B_TOTAL,), jnp.float32),
            jax.ShapeDtypeStruct((B_TOTAL,), jnp.float32),
        ),
        mesh=mesh,
        compiler_params=pltpu.CompilerParams(needs_layout_passes=False),
        scratch_types=[
            pltpu.VMEM((NCHUNK, CHUNK), jnp.int32),
            pltpu.VMEM((NCHUNK, CHUNK), jnp.int32),
            pltpu.VMEM((NCHUNK, CHUNK), jnp.int32),
            pltpu.VMEM((BPW,), jnp.float32),
            pltpu.VMEM((BPW,), jnp.float32),
            pltpu.VMEM((BPW,), jnp.float32),
            pltpu.VMEM((BPW,), jnp.float32),
            pltpu.VMEM((BPW,), jnp.float32),
            pltpu.VMEM((BPW,), jnp.float32),
            pltpu.VMEM((BPW,), jnp.float32),
            pltpu.VMEM((BPW,), jnp.float32),
            pltpu.SemaphoreType.DMA,
        ],
        name="transe_sc",
    )
    o0, o1 = f(e0, e1, r0, r1, head3, rel3, tail3)
    return jnp.stack([o0, o1], axis=1)


def kernel(head, rel, tail, ent_emb, rel_emb, lin_W, lin_b):
    head3 = head.astype(jnp.int32).reshape(NW, NCHUNK, CHUNK)
    rel3 = rel.astype(jnp.int32).reshape(NW, NCHUNK, CHUNK)
    tail3 = tail.astype(jnp.int32).reshape(NW, NCHUNK, CHUNK)
    return _run(head3, rel3, tail3, ent_emb, rel_emb, lin_W, lin_b)


# speedup vs baseline: 2.0223x; 1.0057x over previous
import jax
import jax.numpy as jnp
from jax import lax
from jax.experimental import pallas as pl
from jax.experimental.pallas import tpu as pltpu, tpu_sc as plsc

NC = 2
NS = 16
NW = NC * NS
B_TOTAL = 16384
DIM = 64
NUM_ENTS_C = 1000000
NUM_RELS_C = 1000
BPW = B_TOTAL // NW
CHUNK = 128
NCHUNK = BPW // CHUNK
GROUPS = BPW // 16
BLK = 32768


def _proj_body(wt_ref, tab_ref, rel_ref, b_ref, o0_ref, o1_ref,
               r0_ref, r1_ref):
    res = jnp.dot(wt_ref[...], tab_ref[...],
                  preferred_element_type=jnp.float32)
    o0_ref[...] = res[0]
    o1_ref[...] = res[1]

    @pl.when(pl.program_id(0) == 0)
    def _():
        rp = jnp.dot(wt_ref[...], rel_ref[...],
                     preferred_element_type=jnp.float32) + b_ref[...]
        r0_ref[...] = rp[0]
        r1_ref[...] = rp[1]


def _sc_body(e0_h, e1_h, r0_h, r1_h, head_h, rel_h, tail_h, o0_h, o1_h,
             idx_h, idx_r, idx_t, hp0, hp1, tp0, tp1, rp0, rp1,
             ov0, ov1, sem):
    c = lax.axis_index("c")
    s = lax.axis_index("s")
    wid = s * NC + c
    base = wid * BPW

    pltpu.sync_copy(head_h.at[wid], idx_h)
    pltpu.sync_copy(rel_h.at[wid], idx_r)
    pltpu.sync_copy(tail_h.at[wid], idx_t)

    cps = []
    for k in range(NCHUNK):
        sl = pl.ds(k * CHUNK, CHUNK)
        cps.append(pltpu.async_copy(e0_h.at[idx_h.at[k]], hp0.at[sl], sem))
        cps.append(pltpu.async_copy(e1_h.at[idx_h.at[k]], hp1.at[sl], sem))
        cps.append(pltpu.async_copy(e0_h.at[idx_t.at[k]], tp0.at[sl], sem))
        cps.append(pltpu.async_copy(e1_h.at[idx_t.at[k]], tp1.at[sl], sem))
        cps.append(pltpu.async_copy(r0_h.at[idx_r.at[k]], rp0.at[sl], sem))
        cps.append(pltpu.async_copy(r1_h.at[idx_r.at[k]], rp1.at[sl], sem))
    for cp in cps:
        cp.wait()

    def group_step(g, carry):
        sl = pl.ds(pl.multiple_of(g * 16, 16), 16)
        a0 = hp0[sl] + rp0[sl] - tp0[sl]
        a1 = hp1[sl] + rp1[sl] - tp1[sl]
        ov0[sl] = 1.0 / (1.0 + jnp.exp(-a0))
        ov1[sl] = 1.0 / (1.0 + jnp.exp(-a1))
        return carry

    lax.fori_loop(0, GROUPS, group_step, 0)
    pltpu.sync_copy(ov0, o0_h.at[pl.ds(base, BPW)])
    pltpu.sync_copy(ov1, o1_h.at[pl.ds(base, BPW)])


@jax.jit
def _run(head3, rel3, tail3, ent_emb, rel_emb, lin_W, lin_b):
    wt = lin_W.T
    ent_t = ent_emb.T
    rel_t = rel_emb.T
    b2 = lin_b.reshape(2, 1)

    grid = (NUM_ENTS_C + BLK - 1) // BLK
    e0, e1, r0, r1 = pl.pallas_call(
        _proj_body,
        grid=(grid,),
        in_specs=[
            pl.BlockSpec((2, DIM), lambda i: (0, 0)),
            pl.BlockSpec((DIM, BLK), lambda i: (0, i)),
            pl.BlockSpec((DIM, NUM_RELS_C), lambda i: (0, 0)),
            pl.BlockSpec((2, 1), lambda i: (0, 0)),
        ],
        out_specs=[
            pl.BlockSpec((BLK,), lambda i: (i,)),
            pl.BlockSpec((BLK,), lambda i: (i,)),
            pl.BlockSpec((NUM_RELS_C,), lambda i: (0,)),
            pl.BlockSpec((NUM_RELS_C,), lambda i: (0,)),
        ],
        out_shape=[
            jax.ShapeDtypeStruct((NUM_ENTS_C,), jnp.float32),
            jax.ShapeDtypeStruct((NUM_ENTS_C,), jnp.float32),
            jax.ShapeDtypeStruct((NUM_RELS_C,), jnp.float32),
            jax.ShapeDtypeStruct((NUM_RELS_C,), jnp.float32),
        ],
        compiler_params=pltpu.CompilerParams(vmem_limit_bytes=60000000),
    )(wt, ent_t, rel_t, b2)

    mesh = plsc.VectorSubcoreMesh(core_axis_name="c", subcore_axis_name="s")
    f = pl.kernel(
        _sc_body,
        out_type=(
            jax.ShapeDtypeStruct((B_TOTAL,), jnp.float32),
            jax.ShapeDtypeStruct((B_TOTAL,), jnp.float32),
        ),
        mesh=mesh,
        compiler_params=pltpu.CompilerParams(needs_layout_passes=False),
        scratch_types=[
            pltpu.VMEM((NCHUNK, CHUNK), jnp.int32),
            pltpu.VMEM((NCHUNK, CHUNK), jnp.int32),
            pltpu.VMEM((NCHUNK, CHUNK), jnp.int32),
            pltpu.VMEM((BPW,), jnp.float32),
            pltpu.VMEM((BPW,), jnp.float32),
            pltpu.VMEM((BPW,), jnp.float32),
            pltpu.VMEM((BPW,), jnp.float32),
            pltpu.VMEM((BPW,), jnp.float32),
            pltpu.VMEM((BPW,), jnp.float32),
            pltpu.VMEM((BPW,), jnp.float32),
            pltpu.VMEM((BPW,), jnp.float32),
            pltpu.SemaphoreType.DMA,
        ],
        name="transe_sc",
    )
    o0, o1 = f(e0, e1, r0, r1, head3, rel3, tail3)
    return jnp.stack([o0, o1], axis=1)


def kernel(head, rel, tail, ent_emb, rel_emb, lin_W, lin_b):
    head3 = head.astype(jnp.int32).reshape(NW, NCHUNK, CHUNK)
    rel3 = rel.astype(jnp.int32).reshape(NW, NCHUNK, CHUNK)
    tail3 = tail.astype(jnp.int32).reshape(NW, NCHUNK, CHUNK)
    return _run(head3, rel3, tail3, ent_emb, rel_emb, lin_W, lin_b)


# speedup vs baseline: 2.0282x; 1.0029x over previous
import jax
import jax.numpy as jnp
from jax import lax
from jax.experimental import pallas as pl
from jax.experimental.pallas import tpu as pltpu, tpu_sc as plsc

NC = 2
NS = 16
NW = NC * NS
B_TOTAL = 16384
DIM = 64
NUM_ENTS_C = 1000000
NUM_RELS_C = 1000
BPW = B_TOTAL // NW
CHUNK = 128
NCHUNK = BPW // CHUNK
GROUPS = BPW // 16
BLK = 32768


def _proj_body(wt_ref, tab_ref, rel_ref, b_ref, o0_ref, o1_ref,
               r0_ref, r1_ref):
    res = jnp.dot(wt_ref[...], tab_ref[...],
                  preferred_element_type=jnp.float32)
    o0_ref[...] = res[0]
    o1_ref[...] = res[1]

    @pl.when(pl.program_id(0) == 0)
    def _():
        rp = jnp.dot(wt_ref[...], rel_ref[...],
                     preferred_element_type=jnp.float32) + b_ref[...]
        r0_ref[...] = rp[0]
        r1_ref[...] = rp[1]


def _sc_body(e0_h, e1_h, r0_h, r1_h, head_h, rel_h, tail_h, o0_h, o1_h,
             idx_h, idx_r, idx_t, hp0, hp1, tp0, tp1, rp0, rp1,
             ov0, ov1, sem):
    c = lax.axis_index("c")
    s = lax.axis_index("s")
    wid = s * NC + c
    base = wid * BPW

    pltpu.sync_copy(head_h.at[wid], idx_h)
    pltpu.sync_copy(rel_h.at[wid], idx_r)
    pltpu.sync_copy(tail_h.at[wid], idx_t)

    cps = []
    for k in range(NCHUNK):
        sl = pl.ds(k * CHUNK, CHUNK)
        cps.append(pltpu.async_copy(e0_h.at[idx_h.at[k]], hp0.at[sl], sem))
        cps.append(pltpu.async_copy(e1_h.at[idx_h.at[k]], hp1.at[sl], sem))
        cps.append(pltpu.async_copy(e0_h.at[idx_t.at[k]], tp0.at[sl], sem))
        cps.append(pltpu.async_copy(e1_h.at[idx_t.at[k]], tp1.at[sl], sem))
        cps.append(pltpu.async_copy(r0_h.at[idx_r.at[k]], rp0.at[sl], sem))
        cps.append(pltpu.async_copy(r1_h.at[idx_r.at[k]], rp1.at[sl], sem))

    def group_step(g, carry):
        sl = pl.ds(pl.multiple_of(g * 16, 16), 16)
        a0 = hp0[sl] + rp0[sl] - tp0[sl]
        a1 = hp1[sl] + rp1[sl] - tp1[sl]
        ov0[sl] = 1.0 / (1.0 + jnp.exp(-a0))
        ov1[sl] = 1.0 / (1.0 + jnp.exp(-a1))
        return carry

    for k in range(NCHUNK):
        for cp in cps[6 * k:6 * (k + 1)]:
            cp.wait()
        lax.fori_loop(k * (CHUNK // 16), (k + 1) * (CHUNK // 16),
                      group_step, 0)
        sl = pl.ds(k * CHUNK, CHUNK)
        pltpu.sync_copy(ov0.at[sl], o0_h.at[pl.ds(base + k * CHUNK, CHUNK)])
        pltpu.sync_copy(ov1.at[sl], o1_h.at[pl.ds(base + k * CHUNK, CHUNK)])


@jax.jit
def _run(head3, rel3, tail3, ent_emb, rel_emb, lin_W, lin_b):
    wt = lin_W.T
    ent_t = ent_emb.T
    rel_t = rel_emb.T
    b2 = lin_b.reshape(2, 1)

    grid = (NUM_ENTS_C + BLK - 1) // BLK
    e0, e1, r0, r1 = pl.pallas_call(
        _proj_body,
        grid=(grid,),
        in_specs=[
            pl.BlockSpec((2, DIM), lambda i: (0, 0)),
            pl.BlockSpec((DIM, BLK), lambda i: (0, i)),
            pl.BlockSpec((DIM, NUM_RELS_C), lambda i: (0, 0)),
            pl.BlockSpec((2, 1), lambda i: (0, 0)),
        ],
        out_specs=[
            pl.BlockSpec((BLK,), lambda i: (i,)),
            pl.BlockSpec((BLK,), lambda i: (i,)),
            pl.BlockSpec((NUM_RELS_C,), lambda i: (0,)),
            pl.BlockSpec((NUM_RELS_C,), lambda i: (0,)),
        ],
        out_shape=[
            jax.ShapeDtypeStruct((NUM_ENTS_C,), jnp.float32),
            jax.ShapeDtypeStruct((NUM_ENTS_C,), jnp.float32),
            jax.ShapeDtypeStruct((NUM_RELS_C,), jnp.float32),
            jax.ShapeDtypeStruct((NUM_RELS_C,), jnp.float32),
        ],
        compiler_params=pltpu.CompilerParams(vmem_limit_bytes=60000000),
    )(wt, ent_t, rel_t, b2)

    mesh = plsc.VectorSubcoreMesh(core_axis_name="c", subcore_axis_name="s")
    f = pl.kernel(
        _sc_body,
        out_type=(
            jax.ShapeDtypeStruct((B_TOTAL,), jnp.float32),
            jax.ShapeDtypeStruct((B_TOTAL,), jnp.float32),
        ),
        mesh=mesh,
        compiler_params=pltpu.CompilerParams(needs_layout_passes=False),
        scratch_types=[
            pltpu.VMEM((NCHUNK, CHUNK), jnp.int32),
            pltpu.VMEM((NCHUNK, CHUNK), jnp.int32),
            pltpu.VMEM((NCHUNK, CHUNK), jnp.int32),
            pltpu.VMEM((BPW,), jnp.float32),
            pltpu.VMEM((BPW,), jnp.float32),
            pltpu.VMEM((BPW,), jnp.float32),
            pltpu.VMEM((BPW,), jnp.float32),
            pltpu.VMEM((BPW,), jnp.float32),
            pltpu.VMEM((BPW,), jnp.float32),
            pltpu.VMEM((BPW,), jnp.float32),
            pltpu.VMEM((BPW,), jnp.float32),
            pltpu.SemaphoreType.DMA,
        ],
        name="transe_sc",
    )
    o0, o1 = f(e0, e1, r0, r1, head3, rel3, tail3)
    return jnp.stack([o0, o1], axis=1)


def kernel(head, rel, tail, ent_emb, rel_emb, lin_W, lin_b):
    head3 = head.astype(jnp.int32).reshape(NW, NCHUNK, CHUNK)
    rel3 = rel.astype(jnp.int32).reshape(NW, NCHUNK, CHUNK)
    tail3 = tail.astype(jnp.int32).reshape(NW, NCHUNK, CHUNK)
    return _run(head3, rel3, tail3, ent_emb, rel_emb, lin_W, lin_b)
